# baseline (speedup 1.0000x reference)
import functools

import jax
import jax.numpy as jnp
from jax import lax
from jax.experimental import pallas as pl
from jax.experimental.pallas import tpu as pltpu
from jax.experimental.pallas import tpu_sc as plsc

B, L = 4096, 200
EMB, HID, NCLS = 64, 128, 2
NW = 32
BAGS_PER_W = B // NW
HALF = L // 2
HPAD = 112
NLANE = 16

_mesh = plsc.VectorSubcoreMesh(core_axis_name="c", subcore_axis_name="s")


@functools.partial(
    pl.kernel,
    out_type=jax.ShapeDtypeStruct((B * EMB,), jnp.float32),
    mesh=_mesh,
    scratch_types=[
        pltpu.VMEM((2 * BAGS_PER_W * HPAD,), jnp.int32),
        pltpu.VMEM((2 * BAGS_PER_W * HPAD,), jnp.int32),
        pltpu.VMEM((HALF, 2 * EMB), jnp.float32),
        pltpu.VMEM((HALF, 2 * EMB), jnp.float32),
        pltpu.VMEM((HALF, 2 * EMB), jnp.float32),
        pltpu.VMEM((HALF, 2 * EMB), jnp.float32),
        pltpu.VMEM((BAGS_PER_W * EMB,), jnp.float32),
        pltpu.SemaphoreType.DMA,
        pltpu.SemaphoreType.DMA,
    ],
)
def _embbag_mean(idx_hbm, table_hbm, out_hbm, raw_v, p_v, r0a, r0b, r1a, r1b,
                 outbuf, sem0, sem1):
    wid = lax.axis_index("s") * 2 + lax.axis_index("c")
    base = wid * BAGS_PER_W
    pltpu.sync_copy(idx_hbm.at[pl.ds(2 * base * HPAD, 2 * BAGS_PER_W * HPAD)], raw_v)

    def shift_body(i, carry):
        p_v[pl.ds(i * NLANE, NLANE)] = lax.shift_right_logical(
            raw_v[pl.ds(i * NLANE, NLANE)], 1)
        return carry

    lax.fori_loop(0, 2 * BAGS_PER_W * HPAD // NLANE, shift_body, 0)

    def gather(bag, ra, rb, sem):
        r = 2 * bag
        pltpu.async_copy(table_hbm.at[p_v.at[pl.ds(r * HPAD, HALF)]], ra, sem)
        pltpu.async_copy(table_hbm.at[p_v.at[pl.ds((r + 1) * HPAD, HALF)]], rb, sem)

    def drain(ra, rb, sem):
        pltpu.make_async_copy(table_hbm.at[p_v.at[pl.ds(0, HALF)]], ra, sem).wait()
        pltpu.make_async_copy(table_hbm.at[p_v.at[pl.ds(0, HALF)]], rb, sem).wait()

    def accumulate(bag, ra, rb):
        ia = 2 * bag
        ib = ia + 1

        def add_rows(rows, irow, base_j, nrows, accs):
            offv = (raw_v[pl.ds(irow * HPAD + base_j, NLANE)] & 1) * EMB
            a0, a1, a2, a3 = accs
            for k in range(nrows):
                j = base_j + k
                off = offv[k]
                a0 = a0 + rows[j, pl.ds(off, NLANE)]
                a1 = a1 + rows[j, pl.ds(off + NLANE, NLANE)]
                a2 = a2 + rows[j, pl.ds(off + 2 * NLANE, NLANE)]
                a3 = a3 + rows[j, pl.ds(off + 3 * NLANE, NLANE)]
            return a0, a1, a2, a3

        def body(g, accs):
            base_j = g * NLANE
            accs = add_rows(ra, ia, base_j, NLANE, accs)
            return add_rows(rb, ib, base_j, NLANE, accs)

        z = jnp.zeros((NLANE,), jnp.float32)
        accs = lax.fori_loop(0, HALF // NLANE, body, (z, z, z, z))
        rem = HALF - (HALF // NLANE) * NLANE
        accs = add_rows(ra, ia, HALF - rem, rem, accs)
        return add_rows(rb, ib, HALF - rem, rem, accs)

    scale = jnp.float32(1.0 / L)

    def store(bag, accs):
        a0, a1, a2, a3 = accs
        outbuf[pl.ds(bag * EMB, NLANE)] = a0 * scale
        outbuf[pl.ds(bag * EMB + NLANE, NLANE)] = a1 * scale
        outbuf[pl.ds(bag * EMB + 2 * NLANE, NLANE)] = a2 * scale
        outbuf[pl.ds(bag * EMB + 3 * NLANE, NLANE)] = a3 * scale

    gather(0, r0a, r0b, sem0)
    gather(1, r1a, r1b, sem1)

    def step(i, carry):
        bag = 2 * i
        drain(r0a, r0b, sem0)
        store(bag, accumulate(bag, r0a, r0b))
        gather(jnp.minimum(bag + 2, BAGS_PER_W - 2), r0a, r0b, sem0)
        drain(r1a, r1b, sem1)
        store(bag + 1, accumulate(bag + 1, r1a, r1b))
        gather(jnp.minimum(bag + 3, BAGS_PER_W - 1), r1a, r1b, sem1)
        return carry

    lax.fori_loop(0, BAGS_PER_W // 2, step, 0)
    drain(r0a, r0b, sem0)
    drain(r1a, r1b, sem1)
    pltpu.sync_copy(outbuf, out_hbm.at[pl.ds(base * EMB, BAGS_PER_W * EMB)])


def _mlp_body(x_ref, w1_ref, b1_ref, w2_ref, b2_ref, o_ref):
    x = x_ref[...]
    h = jnp.dot(x, w1_ref[...], preferred_element_type=jnp.float32) + b1_ref[...]
    h = jnp.maximum(h, 0.0)
    logits = jnp.dot(h, w2_ref[...], preferred_element_type=jnp.float32) + b2_ref[...]
    m = jnp.max(logits, axis=1, keepdims=True)
    e = jnp.exp(logits - m)
    o_ref[...] = e / jnp.sum(e, axis=1, keepdims=True)


_mlp = pl.pallas_call(
    _mlp_body,
    out_shape=jax.ShapeDtypeStruct((B, NCLS), jnp.float32),
)


@jax.jit
def kernel(batch_inputs, batch_lengths, emb_table, W1, b1, W2, b2):
    del batch_lengths
    idx = batch_inputs.astype(jnp.int32).reshape(B, 2, HALF)
    idx = jnp.pad(idx, ((0, 0), (0, 0), (0, HPAD - HALF))).reshape(2 * B * HPAD)
    table2 = emb_table.reshape(emb_table.shape[0] // 2, 2 * EMB)
    pooled = _embbag_mean(idx, table2).reshape(B, EMB)
    return _mlp(pooled, W1.T, b1.reshape(1, HID), W2.T, b2.reshape(1, NCLS))


# speedup vs baseline: 1.4211x; 1.4211x over previous
import functools

import jax
import jax.numpy as jnp
from jax import lax
from jax.experimental import pallas as pl
from jax.experimental.pallas import tpu as pltpu
from jax.experimental.pallas import tpu_sc as plsc

B, L = 4096, 200
EMB, HID, NCLS = 64, 128, 2
NW = 32
BAGS_PER_W = B // NW
HALF = L // 2
HPAD = 112
NLANE = 16

_mesh = plsc.VectorSubcoreMesh(core_axis_name="c", subcore_axis_name="s")


@functools.partial(
    pl.kernel,
    out_type=jax.ShapeDtypeStruct((B * EMB,), jnp.float32),
    mesh=_mesh,
    scratch_types=[
        pltpu.VMEM((2 * BAGS_PER_W * HPAD,), jnp.int32),
        pltpu.VMEM((2 * BAGS_PER_W * HPAD,), jnp.int32),
        pltpu.VMEM((HALF, 2 * EMB), jnp.float32),
        pltpu.VMEM((HALF, 2 * EMB), jnp.float32),
        pltpu.VMEM((HALF, 2 * EMB), jnp.float32),
        pltpu.VMEM((HALF, 2 * EMB), jnp.float32),
        pltpu.VMEM((BAGS_PER_W * EMB,), jnp.float32),
        pltpu.SemaphoreType.DMA,
        pltpu.SemaphoreType.DMA,
    ],
)
def _embbag_mean(idx_hbm, table_hbm, out_hbm, raw_v, p_v, r0a, r0b, r1a, r1b,
                 outbuf, sem0, sem1):
    wid = lax.axis_index("s") * 2 + lax.axis_index("c")
    base = wid * BAGS_PER_W
    pltpu.sync_copy(idx_hbm.at[pl.ds(2 * base * HPAD, 2 * BAGS_PER_W * HPAD)], raw_v)

    def shift_body(i, carry):
        v = raw_v[pl.ds(i * NLANE, NLANE)]
        p_v[pl.ds(i * NLANE, NLANE)] = (
            lax.shift_left(lax.shift_right_logical(v, 12), 11) | (v & 2047))
        return carry

    lax.fori_loop(0, 2 * BAGS_PER_W * HPAD // NLANE, shift_body, 0)

    def gather(bag, ra, rb, sem):
        r = 2 * bag
        pltpu.async_copy(table_hbm.at[p_v.at[pl.ds(r * HPAD, HALF)]], ra, sem)
        pltpu.async_copy(table_hbm.at[p_v.at[pl.ds((r + 1) * HPAD, HALF)]], rb, sem)

    def drain(ra, rb, sem):
        pltpu.make_async_copy(table_hbm.at[p_v.at[pl.ds(0, HALF)]], ra, sem).wait()
        pltpu.make_async_copy(table_hbm.at[p_v.at[pl.ds(0, HALF)]], rb, sem).wait()

    def accumulate(bag, ra, rb):
        ia = 2 * bag
        ib = ia + 1

        def add_rows(rows, irow, base_j, nrows, accs):
            offv = (lax.shift_right_logical(
                raw_v[pl.ds(irow * HPAD + base_j, NLANE)], 11) & 1) * EMB
            a0, a1, a2, a3 = accs
            for k in range(nrows):
                j = base_j + k
                off = offv[k]
                a0 = a0 + rows[j, pl.ds(off, NLANE)]
                a1 = a1 + rows[j, pl.ds(off + NLANE, NLANE)]
                a2 = a2 + rows[j, pl.ds(off + 2 * NLANE, NLANE)]
                a3 = a3 + rows[j, pl.ds(off + 3 * NLANE, NLANE)]
            return a0, a1, a2, a3

        def body(g, accs):
            base_j = g * NLANE
            accs = add_rows(ra, ia, base_j, NLANE, accs)
            return add_rows(rb, ib, base_j, NLANE, accs)

        z = jnp.zeros((NLANE,), jnp.float32)
        accs = lax.fori_loop(0, HALF // NLANE, body, (z, z, z, z))
        rem = HALF - (HALF // NLANE) * NLANE
        accs = add_rows(ra, ia, HALF - rem, rem, accs)
        return add_rows(rb, ib, HALF - rem, rem, accs)

    scale = jnp.float32(1.0 / L)

    def store(bag, accs):
        a0, a1, a2, a3 = accs
        outbuf[pl.ds(bag * EMB, NLANE)] = a0 * scale
        outbuf[pl.ds(bag * EMB + NLANE, NLANE)] = a1 * scale
        outbuf[pl.ds(bag * EMB + 2 * NLANE, NLANE)] = a2 * scale
        outbuf[pl.ds(bag * EMB + 3 * NLANE, NLANE)] = a3 * scale

    gather(0, r0a, r0b, sem0)
    gather(1, r1a, r1b, sem1)

    def step(i, carry):
        bag = 2 * i
        drain(r0a, r0b, sem0)
        store(bag, accumulate(bag, r0a, r0b))
        gather(jnp.minimum(bag + 2, BAGS_PER_W - 2), r0a, r0b, sem0)
        drain(r1a, r1b, sem1)
        store(bag + 1, accumulate(bag + 1, r1a, r1b))
        gather(jnp.minimum(bag + 3, BAGS_PER_W - 1), r1a, r1b, sem1)
        return carry

    lax.fori_loop(0, BAGS_PER_W // 2, step, 0)
    drain(r0a, r0b, sem0)
    drain(r1a, r1b, sem1)
    pltpu.sync_copy(outbuf, out_hbm.at[pl.ds(base * EMB, BAGS_PER_W * EMB)])


VB = 4096
PACK_ROWS = VB // 2


def _pack_body(t_ref, o_ref):
    y = t_ref[...].T
    o_ref[...] = jnp.concatenate([y[:PACK_ROWS], y[PACK_ROWS:]], axis=1)


def _pack_table(table_t, vocab):
    nblk = (vocab // 2 + PACK_ROWS - 1) // PACK_ROWS
    return pl.pallas_call(
        _pack_body,
        grid=(nblk,),
        in_specs=[pl.BlockSpec((EMB, VB), lambda j: (0, j))],
        out_specs=pl.BlockSpec((PACK_ROWS, 2 * EMB), lambda j: (j, 0)),
        out_shape=jax.ShapeDtypeStruct((nblk * PACK_ROWS, 2 * EMB), jnp.float32),
    )(table_t)


def _mlp_body(x_ref, w1_ref, b1_ref, w2_ref, b2_ref, o_ref):
    x = x_ref[...]
    h = jnp.dot(x, w1_ref[...], preferred_element_type=jnp.float32) + b1_ref[...]
    h = jnp.maximum(h, 0.0)
    logits = jnp.dot(h, w2_ref[...], preferred_element_type=jnp.float32) + b2_ref[...]
    m = jnp.max(logits, axis=1, keepdims=True)
    e = jnp.exp(logits - m)
    o_ref[...] = e / jnp.sum(e, axis=1, keepdims=True)


_mlp = pl.pallas_call(
    _mlp_body,
    out_shape=jax.ShapeDtypeStruct((B, NCLS), jnp.float32),
)


@jax.jit
def kernel(batch_inputs, batch_lengths, emb_table, W1, b1, W2, b2):
    del batch_lengths
    idx = batch_inputs.astype(jnp.int32).reshape(B, 2, HALF)
    idx = jnp.pad(idx, ((0, 0), (0, 0), (0, HPAD - HALF))).reshape(2 * B * HPAD)
    table2 = _pack_table(emb_table.T, emb_table.shape[0])
    pooled = _embbag_mean(idx, table2).reshape(B, EMB)
    return _mlp(pooled, W1.T, b1.reshape(1, HID), W2.T, b2.reshape(1, NCLS))


# speedup vs baseline: 1.6078x; 1.1314x over previous
import functools

import jax
import jax.numpy as jnp
from jax import lax
from jax.experimental import pallas as pl
from jax.experimental.pallas import tpu as pltpu
from jax.experimental.pallas import tpu_sc as plsc

B, L = 4096, 200
EMB, HID, NCLS = 64, 128, 2
NW = 32
BAGS_PER_W = B // NW
HALF = L // 2
HPAD = 112
NLANE = 16

_mesh = plsc.VectorSubcoreMesh(core_axis_name="c", subcore_axis_name="s")


@functools.partial(
    pl.kernel,
    out_type=jax.ShapeDtypeStruct((B * EMB,), jnp.float32),
    mesh=_mesh,
    scratch_types=[
        pltpu.VMEM((2 * BAGS_PER_W * HPAD,), jnp.int32),
        pltpu.VMEM((2 * BAGS_PER_W * HPAD,), jnp.int32),
        pltpu.VMEM((HALF, 2 * EMB), jnp.float32),
        pltpu.VMEM((HALF, 2 * EMB), jnp.float32),
        pltpu.VMEM((HALF, 2 * EMB), jnp.float32),
        pltpu.VMEM((HALF, 2 * EMB), jnp.float32),
        pltpu.VMEM((BAGS_PER_W * EMB,), jnp.float32),
        pltpu.SemaphoreType.DMA,
        pltpu.SemaphoreType.DMA,
    ],
)
def _embbag_mean(idx_hbm, table_hbm, out_hbm, raw_v, p_v, r0a, r0b, r1a, r1b,
                 outbuf, sem0, sem1):
    wid = lax.axis_index("s") * 2 + lax.axis_index("c")
    base = wid * BAGS_PER_W
    pltpu.sync_copy(idx_hbm.at[pl.ds(2 * base * HPAD, 2 * BAGS_PER_W * HPAD)], raw_v)

    def shift_body(i, carry):
        v = raw_v[pl.ds(i * NLANE, NLANE)]
        p_v[pl.ds(i * NLANE, NLANE)] = (
            lax.shift_left(lax.shift_right_logical(v, 13), 12) | (v & 4095))
        return carry

    lax.fori_loop(0, 2 * BAGS_PER_W * HPAD // NLANE, shift_body, 0)

    def gather(bag, ra, rb, sem):
        r = 2 * bag
        pltpu.async_copy(table_hbm.at[p_v.at[pl.ds(r * HPAD, HALF)]], ra, sem)
        pltpu.async_copy(table_hbm.at[p_v.at[pl.ds((r + 1) * HPAD, HALF)]], rb, sem)

    def drain(ra, rb, sem):
        pltpu.make_async_copy(table_hbm.at[p_v.at[pl.ds(0, HALF)]], ra, sem).wait()
        pltpu.make_async_copy(table_hbm.at[p_v.at[pl.ds(0, HALF)]], rb, sem).wait()

    def accumulate(bag, ra, rb):
        ia = 2 * bag
        ib = ia + 1

        def add_rows(rows, irow, base_j, nrows, accs):
            offv = (lax.shift_right_logical(
                raw_v[pl.ds(irow * HPAD + base_j, NLANE)], 12) & 1) * EMB
            a0, a1, a2, a3 = accs
            for k in range(nrows):
                j = base_j + k
                off = offv[k]
                a0 = a0 + rows[j, pl.ds(off, NLANE)]
                a1 = a1 + rows[j, pl.ds(off + NLANE, NLANE)]
                a2 = a2 + rows[j, pl.ds(off + 2 * NLANE, NLANE)]
                a3 = a3 + rows[j, pl.ds(off + 3 * NLANE, NLANE)]
            return a0, a1, a2, a3

        def body(g, accs):
            base_j = g * NLANE
            accs = add_rows(ra, ia, base_j, NLANE, accs)
            return add_rows(rb, ib, base_j, NLANE, accs)

        z = jnp.zeros((NLANE,), jnp.float32)
        accs = lax.fori_loop(0, HALF // NLANE, body, (z, z, z, z))
        rem = HALF - (HALF // NLANE) * NLANE
        accs = add_rows(ra, ia, HALF - rem, rem, accs)
        return add_rows(rb, ib, HALF - rem, rem, accs)

    scale = jnp.float32(1.0 / L)

    def store(bag, accs):
        a0, a1, a2, a3 = accs
        outbuf[pl.ds(bag * EMB, NLANE)] = a0 * scale
        outbuf[pl.ds(bag * EMB + NLANE, NLANE)] = a1 * scale
        outbuf[pl.ds(bag * EMB + 2 * NLANE, NLANE)] = a2 * scale
        outbuf[pl.ds(bag * EMB + 3 * NLANE, NLANE)] = a3 * scale

    gather(0, r0a, r0b, sem0)
    gather(1, r1a, r1b, sem1)

    def step(i, carry):
        bag = 2 * i
        drain(r0a, r0b, sem0)
        store(bag, accumulate(bag, r0a, r0b))
        gather(jnp.minimum(bag + 2, BAGS_PER_W - 2), r0a, r0b, sem0)
        drain(r1a, r1b, sem1)
        store(bag + 1, accumulate(bag + 1, r1a, r1b))
        gather(jnp.minimum(bag + 3, BAGS_PER_W - 1), r1a, r1b, sem1)
        return carry

    lax.fori_loop(0, BAGS_PER_W // 2, step, 0)
    drain(r0a, r0b, sem0)
    drain(r1a, r1b, sem1)
    pltpu.sync_copy(outbuf, out_hbm.at[pl.ds(base * EMB, BAGS_PER_W * EMB)])


VB = 8192
PACK_ROWS = VB // 2


def _pack_body(t_ref, o_ref):
    y = t_ref[...].T
    o_ref[...] = jnp.concatenate([y[:PACK_ROWS], y[PACK_ROWS:]], axis=1)


def _pack_table(table_t, vocab):
    nblk = (vocab // 2 + PACK_ROWS - 1) // PACK_ROWS
    return pl.pallas_call(
        _pack_body,
        grid=(nblk,),
        in_specs=[pl.BlockSpec((EMB, VB), lambda j: (0, j))],
        out_specs=pl.BlockSpec((PACK_ROWS, 2 * EMB), lambda j: (j, 0)),
        out_shape=jax.ShapeDtypeStruct((nblk * PACK_ROWS, 2 * EMB), jnp.float32),
        compiler_params=pltpu.CompilerParams(fuse_transposed_lhs_in_matmul=True),
    )(table_t)


def _mlp_body(x_ref, w1_ref, b1_ref, w2_ref, b2_ref, o_ref):
    x = x_ref[...]
    h = jnp.dot(x, w1_ref[...], preferred_element_type=jnp.float32) + b1_ref[...]
    h = jnp.maximum(h, 0.0)
    logits = jnp.dot(h, w2_ref[...], preferred_element_type=jnp.float32) + b2_ref[...]
    m = jnp.max(logits, axis=1, keepdims=True)
    e = jnp.exp(logits - m)
    o_ref[...] = e / jnp.sum(e, axis=1, keepdims=True)


_mlp = pl.pallas_call(
    _mlp_body,
    out_shape=jax.ShapeDtypeStruct((B, NCLS), jnp.float32),
)


@jax.jit
def kernel(batch_inputs, batch_lengths, emb_table, W1, b1, W2, b2):
    del batch_lengths
    idx = batch_inputs.astype(jnp.int32).reshape(B, 2, HALF)
    idx = jnp.pad(idx, ((0, 0), (0, 0), (0, HPAD - HALF))).reshape(2 * B * HPAD)
    table2 = _pack_table(emb_table.T, emb_table.shape[0])
    pooled = _embbag_mean(idx, table2).reshape(B, EMB)
    return _mlp(pooled, W1.T, b1.reshape(1, HID), W2.T, b2.reshape(1, NCLS))


# speedup vs baseline: 1.7176x; 1.0683x over previous
import functools

import jax
import jax.numpy as jnp
from jax import lax
from jax.experimental import pallas as pl
from jax.experimental.pallas import tpu as pltpu
from jax.experimental.pallas import tpu_sc as plsc

B, L = 4096, 200
EMB, HID, NCLS = 64, 128, 2
NW = 32
BAGS_PER_W = B // NW
HALF = L // 2
HPAD = 112
NLANE = 16

_mesh = plsc.VectorSubcoreMesh(core_axis_name="c", subcore_axis_name="s")


@functools.partial(
    pl.kernel,
    out_type=jax.ShapeDtypeStruct((B * EMB,), jnp.float32),
    mesh=_mesh,
    scratch_types=[
        pltpu.VMEM((2 * BAGS_PER_W * HPAD,), jnp.int32),
        pltpu.VMEM((2 * BAGS_PER_W * HPAD,), jnp.int32),
        pltpu.VMEM((HALF, 2 * EMB), jnp.float32),
        pltpu.VMEM((HALF, 2 * EMB), jnp.float32),
        pltpu.VMEM((HALF, 2 * EMB), jnp.float32),
        pltpu.VMEM((HALF, 2 * EMB), jnp.float32),
        pltpu.VMEM((BAGS_PER_W * EMB,), jnp.float32),
        pltpu.SemaphoreType.DMA,
        pltpu.SemaphoreType.DMA,
    ],
)
def _embbag_mean(idx_hbm, table_hbm, out_hbm, raw_v, p_v, r0a, r0b, r1a, r1b,
                 outbuf, sem0, sem1):
    wid = lax.axis_index("s") * 2 + lax.axis_index("c")
    base = wid * BAGS_PER_W
    pltpu.sync_copy(idx_hbm.at[pl.ds(2 * base * HPAD, 2 * BAGS_PER_W * HPAD)], raw_v)

    def shift_body(i, carry):
        v = raw_v[pl.ds(i * NLANE, NLANE)]
        p_v[pl.ds(i * NLANE, NLANE)] = (
            lax.shift_left(lax.shift_right_logical(v, 14), 13) | (v & 8191))
        return carry

    lax.fori_loop(0, 2 * BAGS_PER_W * HPAD // NLANE, shift_body, 0)

    def gather(bag, ra, rb, sem):
        r = 2 * bag
        pltpu.async_copy(table_hbm.at[p_v.at[pl.ds(r * HPAD, HALF)]], ra, sem)
        pltpu.async_copy(table_hbm.at[p_v.at[pl.ds((r + 1) * HPAD, HALF)]], rb, sem)

    def drain(ra, rb, sem):
        pltpu.make_async_copy(table_hbm.at[p_v.at[pl.ds(0, HALF)]], ra, sem).wait()
        pltpu.make_async_copy(table_hbm.at[p_v.at[pl.ds(0, HALF)]], rb, sem).wait()

    def accumulate(bag, ra, rb):
        ia = 2 * bag
        ib = ia + 1

        def add_rows(rows, irow, base_j, nrows, accs):
            offv = (lax.shift_right_logical(
                raw_v[pl.ds(irow * HPAD + base_j, NLANE)], 13) & 1) * EMB
            a0, a1, a2, a3 = accs
            for k in range(nrows):
                j = base_j + k
                off = offv[k]
                a0 = a0 + rows[j, pl.ds(off, NLANE)]
                a1 = a1 + rows[j, pl.ds(off + NLANE, NLANE)]
                a2 = a2 + rows[j, pl.ds(off + 2 * NLANE, NLANE)]
                a3 = a3 + rows[j, pl.ds(off + 3 * NLANE, NLANE)]
            return a0, a1, a2, a3

        def body(g, accs):
            base_j = g * NLANE
            accs = add_rows(ra, ia, base_j, NLANE, accs)
            return add_rows(rb, ib, base_j, NLANE, accs)

        z = jnp.zeros((NLANE,), jnp.float32)
        accs = lax.fori_loop(0, HALF // NLANE, body, (z, z, z, z))
        rem = HALF - (HALF // NLANE) * NLANE
        accs = add_rows(ra, ia, HALF - rem, rem, accs)
        return add_rows(rb, ib, HALF - rem, rem, accs)

    scale = jnp.float32(1.0 / L)

    def store(bag, accs):
        a0, a1, a2, a3 = accs
        outbuf[pl.ds(bag * EMB, NLANE)] = a0 * scale
        outbuf[pl.ds(bag * EMB + NLANE, NLANE)] = a1 * scale
        outbuf[pl.ds(bag * EMB + 2 * NLANE, NLANE)] = a2 * scale
        outbuf[pl.ds(bag * EMB + 3 * NLANE, NLANE)] = a3 * scale

    gather(0, r0a, r0b, sem0)
    gather(1, r1a, r1b, sem1)

    def step(i, carry):
        bag = 2 * i
        drain(r0a, r0b, sem0)
        store(bag, accumulate(bag, r0a, r0b))
        gather(jnp.minimum(bag + 2, BAGS_PER_W - 2), r0a, r0b, sem0)
        drain(r1a, r1b, sem1)
        store(bag + 1, accumulate(bag + 1, r1a, r1b))
        gather(jnp.minimum(bag + 3, BAGS_PER_W - 1), r1a, r1b, sem1)
        return carry

    lax.fori_loop(0, BAGS_PER_W // 2, step, 0)
    drain(r0a, r0b, sem0)
    drain(r1a, r1b, sem1)
    pltpu.sync_copy(outbuf, out_hbm.at[pl.ds(base * EMB, BAGS_PER_W * EMB)])


VB = 16384
PACK_ROWS = VB // 2


def _pack_body(t_ref, o_ref):
    y = t_ref[...].T
    o_ref[...] = jnp.concatenate([y[:PACK_ROWS], y[PACK_ROWS:]], axis=1)


def _pack_table(table_t, vocab):
    nblk = (vocab // 2 + PACK_ROWS - 1) // PACK_ROWS
    return pl.pallas_call(
        _pack_body,
        grid=(nblk,),
        in_specs=[pl.BlockSpec((EMB, VB), lambda j: (0, j))],
        out_specs=pl.BlockSpec((PACK_ROWS, 2 * EMB), lambda j: (j, 0)),
        out_shape=jax.ShapeDtypeStruct((nblk * PACK_ROWS, 2 * EMB), jnp.float32),
        compiler_params=pltpu.CompilerParams(fuse_transposed_lhs_in_matmul=True),
    )(table_t)


def _mlp_body(x_ref, w1_ref, b1_ref, w2_ref, b2_ref, o_ref):
    x = x_ref[...]
    h = jnp.dot(x, w1_ref[...], preferred_element_type=jnp.float32) + b1_ref[...]
    h = jnp.maximum(h, 0.0)
    logits = jnp.dot(h, w2_ref[...], preferred_element_type=jnp.float32) + b2_ref[...]
    m = jnp.max(logits, axis=1, keepdims=True)
    e = jnp.exp(logits - m)
    o_ref[...] = e / jnp.sum(e, axis=1, keepdims=True)


_mlp = pl.pallas_call(
    _mlp_body,
    out_shape=jax.ShapeDtypeStruct((B, NCLS), jnp.float32),
)


@jax.jit
def kernel(batch_inputs, batch_lengths, emb_table, W1, b1, W2, b2):
    del batch_lengths
    idx = batch_inputs.astype(jnp.int32).reshape(B, 2, HALF)
    idx = jnp.pad(idx, ((0, 0), (0, 0), (0, HPAD - HALF))).reshape(2 * B * HPAD)
    table2 = _pack_table(emb_table.T, emb_table.shape[0])
    pooled = _embbag_mean(idx, table2).reshape(B, EMB)
    return _mlp(pooled, W1.T, b1.reshape(1, HID), W2.T, b2.reshape(1, NCLS))


# speedup vs baseline: 1.8388x; 1.0705x over previous
import functools

import jax
import jax.numpy as jnp
from jax import lax
from jax.experimental import pallas as pl
from jax.experimental.pallas import tpu as pltpu
from jax.experimental.pallas import tpu_sc as plsc

B, L = 4096, 200
EMB, HID, NCLS = 64, 128, 2
NW = 32
BAGS_PER_W = B // NW
HALF = L // 2
HPAD = 112
NLANE = 16

_mesh = plsc.VectorSubcoreMesh(core_axis_name="c", subcore_axis_name="s")


@functools.partial(
    pl.kernel,
    out_type=jax.ShapeDtypeStruct((B * EMB,), jnp.float32),
    mesh=_mesh,
    scratch_types=[
        pltpu.VMEM((2 * BAGS_PER_W * HPAD,), jnp.int32),
        pltpu.VMEM((2 * BAGS_PER_W * HPAD,), jnp.int32),
        pltpu.VMEM((HALF, 2 * EMB), jnp.float32),
        pltpu.VMEM((HALF, 2 * EMB), jnp.float32),
        pltpu.VMEM((HALF, 2 * EMB), jnp.float32),
        pltpu.VMEM((HALF, 2 * EMB), jnp.float32),
        pltpu.VMEM((BAGS_PER_W * EMB,), jnp.float32),
        pltpu.SemaphoreType.DMA,
        pltpu.SemaphoreType.DMA,
    ],
)
def _embbag_mean(idx_hbm, table_hbm, out_hbm, raw_v, p_v, r0a, r0b, r1a, r1b,
                 outbuf, sem0, sem1):
    wid = lax.axis_index("s") * 2 + lax.axis_index("c")
    base = wid * BAGS_PER_W
    pltpu.sync_copy(idx_hbm.at[pl.ds(2 * base * HPAD, 2 * BAGS_PER_W * HPAD)], raw_v)

    def shift_body(i, carry):
        v = raw_v[pl.ds(i * NLANE, NLANE)]
        p_v[pl.ds(i * NLANE, NLANE)] = (
            lax.shift_left(lax.shift_right_logical(v, 15), 14) | (v & 16383))
        return carry

    lax.fori_loop(0, 2 * BAGS_PER_W * HPAD // NLANE, shift_body, 0)

    def gather(bag, ra, rb, sem):
        r = 2 * bag
        pltpu.async_copy(table_hbm.at[p_v.at[pl.ds(r * HPAD, HALF)]], ra, sem)
        pltpu.async_copy(table_hbm.at[p_v.at[pl.ds((r + 1) * HPAD, HALF)]], rb, sem)

    def drain(ra, rb, sem):
        pltpu.make_async_copy(table_hbm.at[p_v.at[pl.ds(0, HALF)]], ra, sem).wait()
        pltpu.make_async_copy(table_hbm.at[p_v.at[pl.ds(0, HALF)]], rb, sem).wait()

    def accumulate(bag, ra, rb):
        ia = 2 * bag
        ib = ia + 1

        def add_rows(rows, irow, base_j, nrows, accs):
            offv = (lax.shift_right_logical(
                raw_v[pl.ds(irow * HPAD + base_j, NLANE)], 14) & 1) * EMB
            a0, a1, a2, a3 = accs
            for k in range(nrows):
                j = base_j + k
                off = offv[k]
                a0 = a0 + rows[j, pl.ds(off, NLANE)]
                a1 = a1 + rows[j, pl.ds(off + NLANE, NLANE)]
                a2 = a2 + rows[j, pl.ds(off + 2 * NLANE, NLANE)]
                a3 = a3 + rows[j, pl.ds(off + 3 * NLANE, NLANE)]
            return a0, a1, a2, a3

        def body(g, accs):
            base_j = g * NLANE
            accs = add_rows(ra, ia, base_j, NLANE, accs)
            return add_rows(rb, ib, base_j, NLANE, accs)

        z = jnp.zeros((NLANE,), jnp.float32)
        accs = lax.fori_loop(0, HALF // NLANE, body, (z, z, z, z))
        rem = HALF - (HALF // NLANE) * NLANE
        accs = add_rows(ra, ia, HALF - rem, rem, accs)
        return add_rows(rb, ib, HALF - rem, rem, accs)

    scale = jnp.float32(1.0 / L)

    def store(bag, accs):
        a0, a1, a2, a3 = accs
        outbuf[pl.ds(bag * EMB, NLANE)] = a0 * scale
        outbuf[pl.ds(bag * EMB + NLANE, NLANE)] = a1 * scale
        outbuf[pl.ds(bag * EMB + 2 * NLANE, NLANE)] = a2 * scale
        outbuf[pl.ds(bag * EMB + 3 * NLANE, NLANE)] = a3 * scale

    gather(0, r0a, r0b, sem0)
    gather(1, r1a, r1b, sem1)

    def step(i, carry):
        bag = 2 * i
        drain(r0a, r0b, sem0)
        store(bag, accumulate(bag, r0a, r0b))
        gather(jnp.minimum(bag + 2, BAGS_PER_W - 2), r0a, r0b, sem0)
        drain(r1a, r1b, sem1)
        store(bag + 1, accumulate(bag + 1, r1a, r1b))
        gather(jnp.minimum(bag + 3, BAGS_PER_W - 1), r1a, r1b, sem1)
        return carry

    lax.fori_loop(0, BAGS_PER_W // 2, step, 0)
    drain(r0a, r0b, sem0)
    drain(r1a, r1b, sem1)
    pltpu.sync_copy(outbuf, out_hbm.at[pl.ds(base * EMB, BAGS_PER_W * EMB)])


VB = 32768
PACK_ROWS = VB // 2


def _pack_body(t_ref, o_ref):
    y = t_ref[...].T
    o_ref[...] = jnp.concatenate([y[:PACK_ROWS], y[PACK_ROWS:]], axis=1)


def _pack_table(table_t, vocab):
    nblk = (vocab // 2 + PACK_ROWS - 1) // PACK_ROWS
    return pl.pallas_call(
        _pack_body,
        grid=(nblk,),
        in_specs=[pl.BlockSpec((EMB, VB), lambda j: (0, j))],
        out_specs=pl.BlockSpec((PACK_ROWS, 2 * EMB), lambda j: (j, 0)),
        out_shape=jax.ShapeDtypeStruct((nblk * PACK_ROWS, 2 * EMB), jnp.float32),
        compiler_params=pltpu.CompilerParams(fuse_transposed_lhs_in_matmul=True),
    )(table_t)


IB = 1024
IPAD = 2 * HPAD


def _idx_body(x_ref, o_ref):
    y = x_ref[...].T
    z = jnp.zeros((IB, HPAD - HALF), jnp.int32)
    o_ref[...] = jnp.concatenate([y[:, :HALF], z, y[:, HALF:], z], axis=1)


_idx_prep = pl.pallas_call(
    _idx_body,
    grid=(B // IB,),
    in_specs=[pl.BlockSpec((L, IB), lambda j: (0, j))],
    out_specs=pl.BlockSpec((IB, IPAD), lambda j: (j, 0)),
    out_shape=jax.ShapeDtypeStruct((B, IPAD), jnp.int32),
)


def _mlp_body(x_ref, w1_ref, b1_ref, w2_ref, b2_ref, o_ref):
    x = x_ref[...]
    h = jnp.dot(x, w1_ref[...], preferred_element_type=jnp.float32) + b1_ref[...]
    h = jnp.maximum(h, 0.0)
    logits = jnp.dot(h, w2_ref[...], preferred_element_type=jnp.float32) + b2_ref[...]
    m = jnp.max(logits, axis=1, keepdims=True)
    e = jnp.exp(logits - m)
    o_ref[...] = e / jnp.sum(e, axis=1, keepdims=True)


_mlp = pl.pallas_call(
    _mlp_body,
    out_shape=jax.ShapeDtypeStruct((B, NCLS), jnp.float32),
)


@jax.jit
def kernel(batch_inputs, batch_lengths, emb_table, W1, b1, W2, b2):
    del batch_lengths
    idx = _idx_prep(batch_inputs.astype(jnp.int32).T).reshape(2 * B * HPAD)
    table2 = _pack_table(emb_table.T, emb_table.shape[0])
    pooled = _embbag_mean(idx, table2).reshape(B, EMB)
    return _mlp(pooled, W1.T, b1.reshape(1, HID), W2.T, b2.reshape(1, NCLS))


# speedup vs baseline: 1.8580x; 1.0105x over previous
import functools

import jax
import jax.numpy as jnp
from jax import lax
from jax.experimental import pallas as pl
from jax.experimental.pallas import tpu as pltpu
from jax.experimental.pallas import tpu_sc as plsc

B, L = 4096, 200
EMB, HID, NCLS = 64, 128, 2
NW = 32
BAGS_PER_W = B // NW
HALF = L // 2
HPAD = 112
NLANE = 16

_mesh = plsc.VectorSubcoreMesh(core_axis_name="c", subcore_axis_name="s")


@functools.partial(
    pl.kernel,
    out_type=jax.ShapeDtypeStruct((B * EMB,), jnp.float32),
    mesh=_mesh,
    scratch_types=[
        pltpu.VMEM((BAGS_PER_W, 2 * HPAD), jnp.int32),
        pltpu.VMEM((2 * BAGS_PER_W * HPAD,), jnp.int32),
        pltpu.VMEM((HALF, 2 * EMB), jnp.float32),
        pltpu.VMEM((HALF, 2 * EMB), jnp.float32),
        pltpu.VMEM((HALF, 2 * EMB), jnp.float32),
        pltpu.VMEM((HALF, 2 * EMB), jnp.float32),
        pltpu.VMEM((BAGS_PER_W * EMB,), jnp.float32),
        pltpu.SemaphoreType.DMA,
        pltpu.SemaphoreType.DMA,
    ],
)
def _embbag_mean(idx_hbm, table_hbm, out_hbm, raw_v, p_v, r0a, r0b, r1a, r1b,
                 outbuf, sem0, sem1):
    wid = lax.axis_index("s") * 2 + lax.axis_index("c")
    base = wid * BAGS_PER_W
    pltpu.sync_copy(idx_hbm.at[pl.ds(base, BAGS_PER_W)], raw_v)

    def shift_body(i, carry):
        b = i // (2 * HPAD // NLANE)
        c = (i % (2 * HPAD // NLANE)) * NLANE
        v = raw_v[b, pl.ds(c, NLANE)]
        p_v[pl.ds(i * NLANE, NLANE)] = (
            lax.shift_left(lax.shift_right_logical(v, 15), 14) | (v & 16383))
        return carry

    lax.fori_loop(0, 2 * BAGS_PER_W * HPAD // NLANE, shift_body, 0)

    def gather(bag, ra, rb, sem):
        r = 2 * bag
        pltpu.async_copy(table_hbm.at[p_v.at[pl.ds(r * HPAD, HALF)]], ra, sem)
        pltpu.async_copy(table_hbm.at[p_v.at[pl.ds((r + 1) * HPAD, HALF)]], rb, sem)

    def drain(ra, rb, sem):
        pltpu.make_async_copy(table_hbm.at[p_v.at[pl.ds(0, HALF)]], ra, sem).wait()
        pltpu.make_async_copy(table_hbm.at[p_v.at[pl.ds(0, HALF)]], rb, sem).wait()

    def accumulate(bag, ra, rb):
        ia = 2 * bag
        ib = ia + 1

        def add_rows(rows, irow, base_j, nrows, accs):
            offv = (lax.shift_right_logical(
                raw_v[irow // 2, pl.ds((irow % 2) * HPAD + base_j, NLANE)],
                14) & 1) * EMB
            a0, a1, a2, a3 = accs
            for k in range(nrows):
                j = base_j + k
                off = offv[k]
                a0 = a0 + rows[j, pl.ds(off, NLANE)]
                a1 = a1 + rows[j, pl.ds(off + NLANE, NLANE)]
                a2 = a2 + rows[j, pl.ds(off + 2 * NLANE, NLANE)]
                a3 = a3 + rows[j, pl.ds(off + 3 * NLANE, NLANE)]
            return a0, a1, a2, a3

        def body(g, accs):
            base_j = g * NLANE
            accs = add_rows(ra, ia, base_j, NLANE, accs)
            return add_rows(rb, ib, base_j, NLANE, accs)

        z = jnp.zeros((NLANE,), jnp.float32)
        accs = lax.fori_loop(0, HALF // NLANE, body, (z, z, z, z))
        rem = HALF - (HALF // NLANE) * NLANE
        accs = add_rows(ra, ia, HALF - rem, rem, accs)
        return add_rows(rb, ib, HALF - rem, rem, accs)

    scale = jnp.float32(1.0 / L)

    def store(bag, accs):
        a0, a1, a2, a3 = accs
        outbuf[pl.ds(bag * EMB, NLANE)] = a0 * scale
        outbuf[pl.ds(bag * EMB + NLANE, NLANE)] = a1 * scale
        outbuf[pl.ds(bag * EMB + 2 * NLANE, NLANE)] = a2 * scale
        outbuf[pl.ds(bag * EMB + 3 * NLANE, NLANE)] = a3 * scale

    gather(0, r0a, r0b, sem0)
    gather(1, r1a, r1b, sem1)

    def step(i, carry):
        bag = 2 * i
        drain(r0a, r0b, sem0)
        store(bag, accumulate(bag, r0a, r0b))
        gather(jnp.minimum(bag + 2, BAGS_PER_W - 2), r0a, r0b, sem0)
        drain(r1a, r1b, sem1)
        store(bag + 1, accumulate(bag + 1, r1a, r1b))
        gather(jnp.minimum(bag + 3, BAGS_PER_W - 1), r1a, r1b, sem1)
        return carry

    lax.fori_loop(0, BAGS_PER_W // 2, step, 0)
    drain(r0a, r0b, sem0)
    drain(r1a, r1b, sem1)
    pltpu.sync_copy(outbuf, out_hbm.at[pl.ds(base * EMB, BAGS_PER_W * EMB)])


VB = 32768
PACK_ROWS = VB // 2


def _pack_body(t_ref, o_ref):
    y = t_ref[...].T
    o_ref[...] = jnp.concatenate([y[:PACK_ROWS], y[PACK_ROWS:]], axis=1)


def _pack_table(table_t, vocab):
    nblk = (vocab // 2 + PACK_ROWS - 1) // PACK_ROWS
    return pl.pallas_call(
        _pack_body,
        grid=(nblk,),
        in_specs=[pl.BlockSpec((EMB, VB), lambda j: (0, j))],
        out_specs=pl.BlockSpec((PACK_ROWS, 2 * EMB), lambda j: (j, 0)),
        out_shape=jax.ShapeDtypeStruct((nblk * PACK_ROWS, 2 * EMB), jnp.float32),
        compiler_params=pltpu.CompilerParams(fuse_transposed_lhs_in_matmul=True),
    )(table_t)


IB = 1024
IPAD = 2 * HPAD


def _idx_body(x_ref, o_ref):
    y = x_ref[...].T
    z = jnp.zeros((IB, HPAD - HALF), jnp.int32)
    o_ref[...] = jnp.concatenate([y[:, :HALF], z, y[:, HALF:], z], axis=1)


_idx_prep = pl.pallas_call(
    _idx_body,
    grid=(B // IB,),
    in_specs=[pl.BlockSpec((L, IB), lambda j: (0, j))],
    out_specs=pl.BlockSpec((IB, IPAD), lambda j: (j, 0)),
    out_shape=jax.ShapeDtypeStruct((B, IPAD), jnp.int32),
)


def _mlp_body(x_ref, w1_ref, b1_ref, w2_ref, b2_ref, o_ref):
    x = x_ref[...]
    h = jnp.dot(x, w1_ref[...], preferred_element_type=jnp.float32) + b1_ref[...]
    h = jnp.maximum(h, 0.0)
    logits = jnp.dot(h, w2_ref[...], preferred_element_type=jnp.float32) + b2_ref[...]
    m = jnp.max(logits, axis=1, keepdims=True)
    e = jnp.exp(logits - m)
    o_ref[...] = e / jnp.sum(e, axis=1, keepdims=True)


_mlp = pl.pallas_call(
    _mlp_body,
    out_shape=jax.ShapeDtypeStruct((B, NCLS), jnp.float32),
)


@jax.jit
def kernel(batch_inputs, batch_lengths, emb_table, W1, b1, W2, b2):
    del batch_lengths
    idx = _idx_prep(batch_inputs.astype(jnp.int32).T)
    table2 = _pack_table(emb_table.T, emb_table.shape[0])
    pooled = _embbag_mean(idx, table2).reshape(B, EMB)
    return _mlp(pooled, W1.T, b1.reshape(1, HID), W2.T, b2.reshape(1, NCLS))


# speedup vs baseline: 1.9698x; 1.0602x over previous
import functools

import jax
import jax.numpy as jnp
from jax import lax
from jax.experimental import pallas as pl
from jax.experimental.pallas import tpu as pltpu
from jax.experimental.pallas import tpu_sc as plsc

B, L = 4096, 200
EMB, HID, NCLS = 64, 128, 2
NW = 32
BAGS_PER_W = B // NW
HALF = L // 2
HPAD = 112
NLANE = 16

_mesh = plsc.VectorSubcoreMesh(core_axis_name="c", subcore_axis_name="s")


@functools.partial(
    pl.kernel,
    out_type=jax.ShapeDtypeStruct((B * EMB,), jnp.float32),
    mesh=_mesh,
    scratch_types=[
        pltpu.VMEM((BAGS_PER_W, 2 * HPAD), jnp.int32),
        pltpu.VMEM((2 * BAGS_PER_W * HPAD,), jnp.int32),
        pltpu.VMEM((HALF, 2 * EMB), jnp.float32),
        pltpu.VMEM((HALF, 2 * EMB), jnp.float32),
        pltpu.VMEM((HALF, 2 * EMB), jnp.float32),
        pltpu.VMEM((HALF, 2 * EMB), jnp.float32),
        pltpu.VMEM((BAGS_PER_W * EMB,), jnp.float32),
        pltpu.SemaphoreType.DMA,
        pltpu.SemaphoreType.DMA,
        pltpu.SemaphoreType.DMA,
        pltpu.SemaphoreType.DMA,
    ],
)
def _embbag_mean(idx_hbm, table_hbm, out_hbm, raw_v, p_v, r0a, r0b, r1a, r1b,
                 outbuf, sem0, sem1, sem2, sem3):
    wid = lax.axis_index("s") * 2 + lax.axis_index("c")
    base = wid * BAGS_PER_W
    pltpu.sync_copy(idx_hbm.at[pl.ds(base, BAGS_PER_W)], raw_v)

    def shift_body(i, carry):
        b = i // (2 * HPAD // NLANE)
        c = (i % (2 * HPAD // NLANE)) * NLANE
        v = raw_v[b, pl.ds(c, NLANE)]
        p_v[pl.ds(i * NLANE, NLANE)] = (
            lax.shift_left(lax.shift_right_logical(v, 15), 14) | (v & 16383))
        return carry

    lax.fori_loop(0, 2 * BAGS_PER_W * HPAD // NLANE, shift_body, 0)

    def gather(unit, rows, sem):
        pltpu.async_copy(table_hbm.at[p_v.at[pl.ds(unit * HPAD, HALF)]], rows, sem)

    def drain(rows, sem):
        pltpu.make_async_copy(table_hbm.at[p_v.at[pl.ds(0, HALF)]], rows, sem).wait()

    def add_rows(rows, unit, base_j, nrows, accs):
        offv = (lax.shift_right_logical(
            raw_v[unit // 2, pl.ds((unit % 2) * HPAD + base_j, NLANE)],
            14) & 1) * EMB
        a0, a1, a2, a3 = accs
        for k in range(nrows):
            j = base_j + k
            off = offv[k]
            a0 = a0 + rows[j, pl.ds(off, NLANE)]
            a1 = a1 + rows[j, pl.ds(off + NLANE, NLANE)]
            a2 = a2 + rows[j, pl.ds(off + 2 * NLANE, NLANE)]
            a3 = a3 + rows[j, pl.ds(off + 3 * NLANE, NLANE)]
        return a0, a1, a2, a3

    def accumulate_half(rows, unit, accs):
        def body(g, accs):
            return add_rows(rows, unit, g * NLANE, NLANE, accs)

        accs = lax.fori_loop(0, HALF // NLANE, body, accs)
        rem = HALF - (HALF // NLANE) * NLANE
        return add_rows(rows, unit, HALF - rem, rem, accs)

    scale = jnp.float32(1.0 / L)

    def store(bag, accs):
        a0, a1, a2, a3 = accs
        outbuf[pl.ds(bag * EMB, NLANE)] = a0 * scale
        outbuf[pl.ds(bag * EMB + NLANE, NLANE)] = a1 * scale
        outbuf[pl.ds(bag * EMB + 2 * NLANE, NLANE)] = a2 * scale
        outbuf[pl.ds(bag * EMB + 3 * NLANE, NLANE)] = a3 * scale

    bufs = (r0a, r0b, r1a, r1b)
    sems = (sem0, sem1, sem2, sem3)
    nunits = 2 * BAGS_PER_W
    for k in range(4):
        gather(k, bufs[k], sems[k])

    z = jnp.zeros((NLANE,), jnp.float32)

    def step(i, carry):
        for k in range(4):
            u = 4 * i + k
            drain(bufs[k], sems[k])
            if k % 2 == 0:
                accs = accumulate_half(bufs[k], u, (z, z, z, z))
            else:
                accs = accumulate_half(bufs[k], u, accs)
                store(u // 2, accs)
            gather(jnp.minimum(u + 4, nunits - 4 + k), bufs[k], sems[k])
        return carry

    lax.fori_loop(0, BAGS_PER_W // 2, step, 0)
    for k in range(4):
        drain(bufs[k], sems[k])
    pltpu.sync_copy(outbuf, out_hbm.at[pl.ds(base * EMB, BAGS_PER_W * EMB)])


VB = 32768
PACK_ROWS = VB // 2


def _pack_body(t_ref, o_ref):
    y = t_ref[...].T
    o_ref[...] = jnp.concatenate([y[:PACK_ROWS], y[PACK_ROWS:]], axis=1)


def _pack_table(table_t, vocab):
    nblk = (vocab // 2 + PACK_ROWS - 1) // PACK_ROWS
    return pl.pallas_call(
        _pack_body,
        grid=(nblk,),
        in_specs=[pl.BlockSpec((EMB, VB), lambda j: (0, j))],
        out_specs=pl.BlockSpec((PACK_ROWS, 2 * EMB), lambda j: (j, 0)),
        out_shape=jax.ShapeDtypeStruct((nblk * PACK_ROWS, 2 * EMB), jnp.float32),
        compiler_params=pltpu.CompilerParams(fuse_transposed_lhs_in_matmul=True),
    )(table_t)


IB = 1024
IPAD = 2 * HPAD


def _idx_body(x_ref, o_ref):
    y = x_ref[...].T
    z = jnp.zeros((IB, HPAD - HALF), jnp.int32)
    o_ref[...] = jnp.concatenate([y[:, :HALF], z, y[:, HALF:], z], axis=1)


_idx_prep = pl.pallas_call(
    _idx_body,
    grid=(B // IB,),
    in_specs=[pl.BlockSpec((L, IB), lambda j: (0, j))],
    out_specs=pl.BlockSpec((IB, IPAD), lambda j: (j, 0)),
    out_shape=jax.ShapeDtypeStruct((B, IPAD), jnp.int32),
)


def _mlp_body(x_ref, w1_ref, b1_ref, w2_ref, b2_ref, o_ref):
    x = x_ref[...]
    h = jnp.dot(x, w1_ref[...], preferred_element_type=jnp.float32) + b1_ref[...]
    h = jnp.maximum(h, 0.0)
    logits = jnp.dot(h, w2_ref[...], preferred_element_type=jnp.float32) + b2_ref[...]
    m = jnp.max(logits, axis=1, keepdims=True)
    e = jnp.exp(logits - m)
    o_ref[...] = e / jnp.sum(e, axis=1, keepdims=True)


_mlp = pl.pallas_call(
    _mlp_body,
    out_shape=jax.ShapeDtypeStruct((B, NCLS), jnp.float32),
)


@jax.jit
def kernel(batch_inputs, batch_lengths, emb_table, W1, b1, W2, b2):
    del batch_lengths
    idx = _idx_prep(batch_inputs.astype(jnp.int32).T)
    table2 = _pack_table(emb_table.T, emb_table.shape[0])
    pooled = _embbag_mean(idx, table2).reshape(B, EMB)
    return _mlp(pooled, W1.T, b1.reshape(1, HID), W2.T, b2.reshape(1, NCLS))


# speedup vs baseline: 1.9828x; 1.0066x over previous
import functools

import jax
import jax.numpy as jnp
from jax import lax
from jax.experimental import pallas as pl
from jax.experimental.pallas import tpu as pltpu
from jax.experimental.pallas import tpu_sc as plsc

B, L = 4096, 200
EMB, HID, NCLS = 64, 128, 2
NW = 32
BAGS_PER_W = B // NW
HALF = L // 2
HPAD = 112
NLANE = 16

_mesh = plsc.VectorSubcoreMesh(core_axis_name="c", subcore_axis_name="s")


@functools.partial(
    pl.kernel,
    out_type=jax.ShapeDtypeStruct((B, EMB), jnp.float32),
    mesh=_mesh,
    scratch_types=[
        pltpu.VMEM((BAGS_PER_W, 2 * HPAD), jnp.int32),
        pltpu.VMEM((2 * BAGS_PER_W * HPAD,), jnp.int32),
        pltpu.VMEM((HALF, 2 * EMB), jnp.float32),
        pltpu.VMEM((HALF, 2 * EMB), jnp.float32),
        pltpu.VMEM((HALF, 2 * EMB), jnp.float32),
        pltpu.VMEM((HALF, 2 * EMB), jnp.float32),
        pltpu.VMEM((BAGS_PER_W // 2, EMB), jnp.float32),
        pltpu.SemaphoreType.DMA,
        pltpu.SemaphoreType.DMA,
        pltpu.SemaphoreType.DMA,
        pltpu.SemaphoreType.DMA,
    ],
)
def _embbag_mean(idx_hbm, table_hbm, out_hbm, raw_v, p_v, r0a, r0b, r1a, r1b,
                 outbuf, sem0, sem1, sem2, sem3):
    wid = lax.axis_index("s") * 2 + lax.axis_index("c")
    base = wid * BAGS_PER_W
    pltpu.sync_copy(idx_hbm.at[pl.ds(base, BAGS_PER_W)], raw_v)

    def shift_body(i, carry):
        b = i // (2 * HPAD // NLANE)
        c = (i % (2 * HPAD // NLANE)) * NLANE
        v = raw_v[b, pl.ds(c, NLANE)]
        p_v[pl.ds(i * NLANE, NLANE)] = (
            lax.shift_left(lax.shift_right_logical(v, 15), 14) | (v & 16383))
        return carry

    lax.fori_loop(0, 2 * BAGS_PER_W * HPAD // NLANE, shift_body, 0)

    def gather(unit, rows, sem):
        pltpu.async_copy(table_hbm.at[p_v.at[pl.ds(unit * HPAD, HALF)]], rows, sem)

    def drain(rows, sem):
        pltpu.make_async_copy(table_hbm.at[p_v.at[pl.ds(0, HALF)]], rows, sem).wait()

    def add_rows(rows, unit, base_j, nrows, accs):
        offv = (lax.shift_right_logical(
            raw_v[unit // 2, pl.ds((unit % 2) * HPAD + base_j, NLANE)],
            14) & 1) * EMB
        a0, a1, a2, a3 = accs
        for k in range(nrows):
            j = base_j + k
            off = offv[k]
            a0 = a0 + rows[j, pl.ds(off, NLANE)]
            a1 = a1 + rows[j, pl.ds(off + NLANE, NLANE)]
            a2 = a2 + rows[j, pl.ds(off + 2 * NLANE, NLANE)]
            a3 = a3 + rows[j, pl.ds(off + 3 * NLANE, NLANE)]
        return a0, a1, a2, a3

    def accumulate_half(rows, unit, accs):
        def body(g, accs):
            return add_rows(rows, unit, g * NLANE, NLANE, accs)

        accs = lax.fori_loop(0, HALF // NLANE, body, accs)
        rem = HALF - (HALF // NLANE) * NLANE
        return add_rows(rows, unit, HALF - rem, rem, accs)

    scale = jnp.float32(1.0 / L)

    def store(bag, accs):
        a0, a1, a2, a3 = accs
        row = bag % (BAGS_PER_W // 2)
        outbuf[row, pl.ds(0, NLANE)] = a0 * scale
        outbuf[row, pl.ds(NLANE, NLANE)] = a1 * scale
        outbuf[row, pl.ds(2 * NLANE, NLANE)] = a2 * scale
        outbuf[row, pl.ds(3 * NLANE, NLANE)] = a3 * scale

    bufs = (r0a, r0b, r1a, r1b)
    sems = (sem0, sem1, sem2, sem3)
    nunits = 2 * BAGS_PER_W
    for k in range(4):
        gather(k, bufs[k], sems[k])

    z = jnp.zeros((NLANE,), jnp.float32)

    def step(i, carry):
        for k in range(4):
            u = 4 * i + k
            drain(bufs[k], sems[k])
            if k % 2 == 0:
                accs = accumulate_half(bufs[k], u, (z, z, z, z))
            else:
                accs = accumulate_half(bufs[k], u, accs)
                store(u // 2, accs)
            gather(jnp.minimum(u + 4, nunits - 4 + k), bufs[k], sems[k])
        return carry

    lax.fori_loop(0, BAGS_PER_W // 4, step, 0)
    pltpu.sync_copy(outbuf, out_hbm.at[pl.ds(base, BAGS_PER_W // 2)])
    lax.fori_loop(BAGS_PER_W // 4, BAGS_PER_W // 2, step, 0)
    for k in range(4):
        drain(bufs[k], sems[k])
    pltpu.sync_copy(outbuf, out_hbm.at[pl.ds(base + BAGS_PER_W // 2, BAGS_PER_W // 2)])


VB = 32768
PACK_ROWS = VB // 2


def _pack_body(t_ref, o_ref):
    y = t_ref[...].T
    o_ref[...] = jnp.concatenate([y[:PACK_ROWS], y[PACK_ROWS:]], axis=1)


def _pack_table(table_t, vocab):
    nblk = (vocab // 2 + PACK_ROWS - 1) // PACK_ROWS
    return pl.pallas_call(
        _pack_body,
        grid=(nblk,),
        in_specs=[pl.BlockSpec((EMB, VB), lambda j: (0, j))],
        out_specs=pl.BlockSpec((PACK_ROWS, 2 * EMB), lambda j: (j, 0)),
        out_shape=jax.ShapeDtypeStruct((nblk * PACK_ROWS, 2 * EMB), jnp.float32),
    )(table_t)


IB = 1024
IPAD = 2 * HPAD


def _idx_body(x_ref, o_ref):
    y = x_ref[...].T
    z = jnp.zeros((IB, HPAD - HALF), jnp.int32)
    o_ref[...] = jnp.concatenate([y[:, :HALF], z, y[:, HALF:], z], axis=1)


_idx_prep = pl.pallas_call(
    _idx_body,
    grid=(B // IB,),
    in_specs=[pl.BlockSpec((L, IB), lambda j: (0, j))],
    out_specs=pl.BlockSpec((IB, IPAD), lambda j: (j, 0)),
    out_shape=jax.ShapeDtypeStruct((B, IPAD), jnp.int32),
)


def _mlp_body(x_ref, w1_ref, b1_ref, w2_ref, b2_ref, o_ref):
    x = x_ref[...]
    h = jnp.dot(x, w1_ref[...], preferred_element_type=jnp.float32) + b1_ref[...]
    h = jnp.maximum(h, 0.0)
    logits = jnp.dot(h, w2_ref[...], preferred_element_type=jnp.float32) + b2_ref[...]
    m = jnp.max(logits, axis=1, keepdims=True)
    e = jnp.exp(logits - m)
    o_ref[...] = e / jnp.sum(e, axis=1, keepdims=True)


_mlp = pl.pallas_call(
    _mlp_body,
    out_shape=jax.ShapeDtypeStruct((B, NCLS), jnp.float32),
)


@jax.jit
def kernel(batch_inputs, batch_lengths, emb_table, W1, b1, W2, b2):
    del batch_lengths
    idx = _idx_prep(batch_inputs.astype(jnp.int32).T)
    table2 = _pack_table(emb_table.T, emb_table.shape[0])
    pooled = _embbag_mean(idx, table2)
    return _mlp(pooled, W1.T, b1.reshape(1, HID), W2.T, b2.reshape(1, NCLS))
